# baseline (speedup 1.0000x reference)
import functools

import jax
import jax.numpy as jnp
from jax import lax
from jax.experimental import pallas as pl
from jax.experimental.pallas import tpu as pltpu
from jax.experimental.pallas import tpu_sc as plsc

N_TASKS = 12
MAX_ATOMS = 30
N_ATOM_FEAT = 75
NGF = 30
N_OUT = 30
BATCH = 128
N = MAX_ATOMS * BATCH
F = 32
SLOTS = MAX_ATOMS + 1


def _sc_gather(table, idx, chunk):
    info = plsc.get_sparse_core_info()
    nw = info.num_cores * info.num_subcores
    b = idx.shape[0]
    bpw = b // nw
    nch = bpw // chunk
    assert bpw % chunk == 0 and chunk % 8 == 0 and chunk <= 128
    mesh = plsc.VectorSubcoreMesh(core_axis_name="c", subcore_axis_name="s")

    @functools.partial(
        pl.kernel,
        mesh=mesh,
        out_type=jax.ShapeDtypeStruct((b, F), jnp.float32),
        scratch_types=[
            pltpu.VMEM((bpw,), jnp.int32),
            pltpu.VMEM((bpw, F), jnp.float32),
            pltpu.SemaphoreType.DMA,
        ],
        compiler_params=pltpu.CompilerParams(use_tc_tiling_on_sc=False),
    )
    def k(table_hbm, idx_hbm, out_hbm, idx_v, rows_v, sem):
        wid = lax.axis_index("s") * info.num_cores + lax.axis_index("c")
        base = pl.multiple_of(wid * bpw, 8)
        pltpu.sync_copy(idx_hbm.at[pl.ds(base, bpw)], idx_v)

        def body(c, carry):
            off = pl.multiple_of(c * chunk, 8)
            pltpu.async_copy(
                table_hbm.at[idx_v.at[pl.ds(off, chunk)]],
                rows_v.at[pl.ds(off, chunk)],
                sem,
            )
            return carry

        lax.fori_loop(0, nch, body, 0)
        pltpu.make_async_copy(table_hbm.at[pl.ds(0, bpw)], rows_v, sem).wait()
        pltpu.sync_copy(rows_v, out_hbm.at[pl.ds(base, bpw)])

    return k(table, idx)


def _tc_source_rounds(par_flat, colst, lane_t):
    blk = 480
    grid = N // blk
    jt = par_flat.shape[1]

    def body(p_ref, c_ref, lt_ref, o_ref):
        i = pl.program_id(0)
        acc = jnp.full((blk, jt), MAX_ATOMS, jnp.int32)
        p = p_ref[...]
        lt = lt_ref[...]
        for tp in range(MAX_ATOMS):
            m = (p == c_ref[:, tp:tp + 1]) & (lt > tp)
            acc = jnp.where(m, tp, acc)
        rows = lax.broadcasted_iota(jnp.int32, (blk, jt), 0) + i * blk
        o_ref[...] = acc * N + rows

    return pl.pallas_call(
        body,
        grid=(grid,),
        in_specs=[
            pl.BlockSpec((blk, jt), lambda i: (i, 0)),
            pl.BlockSpec((blk, MAX_ATOMS), lambda i: (i, 0)),
            pl.BlockSpec((1, jt), lambda i: (0, 0)),
        ],
        out_specs=pl.BlockSpec((blk, jt), lambda i: (i, 0)),
        out_shape=jax.ShapeDtypeStruct((N, jt), jnp.int32),
    )(par_flat, colst, lane_t)

def _tc_atom_proj(x, w_t, b):
    def body(x_ref, w_ref, b_ref, o_ref):
        o_ref[...] = (
            jnp.dot(x_ref[...], w_ref[...], preferred_element_type=jnp.float32)
            + b_ref[...]
        )

    return pl.pallas_call(
        body,
        out_shape=jax.ShapeDtypeStruct((x.shape[0], F), jnp.float32),
    )(x, w_t, b)


def _tc_round_mlp(hist, gflat, ag3, t, w0g_t, w1_t, b1):
    blk = 480
    grid = N // blk

    def body(g_ref, a_ref, w0_ref, w1_ref, b1_ref, h_ref, o_ref):
        del h_ref
        h = jnp.dot(g_ref[...], w0_ref[...], preferred_element_type=jnp.float32)
        h = jnp.maximum(h + a_ref[0], 0.0)
        o = jnp.dot(h, w1_ref[...], preferred_element_type=jnp.float32)
        o_ref[0] = jnp.maximum(o + b1_ref[...], 0.0)

    return pl.pallas_call(
        body,
        grid=(grid,),
        in_specs=[
            pl.BlockSpec((blk, gflat.shape[1]), lambda i: (i, 0)),
            pl.BlockSpec((1, blk, F), lambda i, t=t: (t, i, 0)),
            pl.BlockSpec(w0g_t.shape, lambda i: (0, 0)),
            pl.BlockSpec(w1_t.shape, lambda i: (0, 0)),
            pl.BlockSpec(b1.shape, lambda i: (0, 0)),
            pl.BlockSpec((1, blk, F), lambda i, t=t: (t, i, 0)),
        ],
        out_specs=pl.BlockSpec((1, blk, F), lambda i, t=t: (t, i, 0)),
        out_shape=jax.ShapeDtypeStruct((SLOTS, N, F), jnp.float32),
        input_output_aliases={5: 0},
    )(gflat, ag3, w0g_t, w1_t, b1, hist)


def _tc_round0(hist, ag3, w1_t, b1):
    blk = 480
    grid = N // blk

    def body(a_ref, w1_ref, b1_ref, h_ref, o_ref):
        del h_ref
        h = jnp.maximum(a_ref[0], 0.0)
        o = jnp.dot(h, w1_ref[...], preferred_element_type=jnp.float32)
        o_ref[0] = jnp.maximum(o + b1_ref[...], 0.0)

    return pl.pallas_call(
        body,
        grid=(grid,),
        in_specs=[
            pl.BlockSpec((1, blk, F), lambda i: (0, i, 0)),
            pl.BlockSpec(w1_t.shape, lambda i: (0, 0)),
            pl.BlockSpec(b1.shape, lambda i: (0, 0)),
            pl.BlockSpec((1, blk, F), lambda i: (0, i, 0)),
        ],
        out_specs=pl.BlockSpec((1, blk, F), lambda i: (0, i, 0)),
        out_shape=jax.ShapeDtypeStruct((SLOTS, N, F), jnp.float32),
        input_output_aliases={3: 0},
    )(ag3, w1_t, b1, hist)


def _tc_head(hist, mem_col, gw0_t, gb0, gw1_t, gb1, dw_t, db, pswap):
    def body(x_ref, m_ref, w0_ref, b0_ref, w1_ref, b1_ref, wd_ref, bd_ref,
             p_ref, soft_ref, logit_ref):
        seg = lax.broadcasted_iota(jnp.int32, (N, BATCH), 1)
        oh = (m_ref[...] == seg).astype(jnp.float32)
        g = lax.dot_general(
            oh, x_ref[0], (((0,), (0,)), ((), ())),
            preferred_element_type=jnp.float32,
        )
        h = jnp.maximum(
            jnp.dot(g, w0_ref[...], preferred_element_type=jnp.float32)
            + b0_ref[...], 0.0)
        h = jnp.maximum(
            jnp.dot(h, w1_ref[...], preferred_element_type=jnp.float32)
            + b1_ref[...], 0.0)
        x = (jnp.dot(h, wd_ref[...], preferred_element_type=jnp.float32)
             + bd_ref[...])
        partner = jnp.dot(x, p_ref[...], preferred_element_type=jnp.float32)
        m = jnp.maximum(x, partner)
        e = jnp.exp(x - m)
        s = e + jnp.exp(partner - m)
        soft_ref[...] = e / s
        logit_ref[...] = x

    def _full(x):
        zero = tuple(0 for _ in x.shape)
        return pl.BlockSpec(x.shape, lambda i, _z=zero: _z)

    specs = [pl.BlockSpec((1, N, F), lambda i: (MAX_ATOMS - 1, 0, 0))]
    specs += [_full(x)
              for x in (mem_col, gw0_t, gb0, gw1_t, gb1, dw_t, db, pswap)]
    oshape = (BATCH, 2 * N_TASKS)
    return pl.pallas_call(
        body,
        grid=(1,),
        in_specs=specs,
        out_specs=(pl.BlockSpec(oshape, lambda i: (0, 0)),
                   pl.BlockSpec(oshape, lambda i: (0, 0))),
        out_shape=(
            jax.ShapeDtypeStruct((BATCH, 2 * N_TASKS), jnp.float32),
            jax.ShapeDtypeStruct((BATCH, 2 * N_TASKS), jnp.float32),
        ),
    )(hist, mem_col, gw0_t, gb0, gw1_t, gb1, dw_t, db, pswap)


def kernel(atom_features, parents, calculation_orders, calculation_masks,
           membership, n_atoms, dag_W0, dag_b0, dag_W1, dag_b1,
           gat_W0, gat_b0, gat_W1, gat_b1, dense_W, dense_b):
    del calculation_masks, n_atoms

    w0a_t = dag_W0[:, :N_ATOM_FEAT].T
    b0 = dag_b0.reshape(1, F)
    w0g = dag_W0[:, N_ATOM_FEAT:].reshape(F, MAX_ATOMS - 1, NGF)
    w0g = jnp.pad(w0g, ((0, 0), (0, 0), (0, F - NGF)))
    w0g_t = w0g.reshape(F, (MAX_ATOMS - 1) * F).T
    w1_t = jnp.pad(dag_W1.T, ((0, 0), (0, F - N_OUT)))
    b1 = jnp.pad(dag_b1, (0, F - N_OUT)).reshape(1, F)
    gw0_t = jnp.pad(gat_W0.T, ((0, F - NGF), (0, 0)))
    gb0 = gat_b0.reshape(1, -1)
    gw1_t = jnp.pad(gat_W1.T, ((0, 0), (0, F - N_OUT)))
    gb1 = jnp.pad(gat_b1, (0, F - N_OUT)).reshape(1, F)
    dw_t = jnp.pad(dense_W.T, ((0, F - N_OUT), (0, 0)))
    db = dense_b.reshape(1, -1)
    ncls = 2 * N_TASKS
    pair = jnp.arange(ncls)
    pswap = (pair[:, None] == (pair ^ 1)[None, :]).astype(jnp.float32)

    par32 = parents.astype(jnp.int32)
    nj = MAX_ATOMS - 1
    par_flat = par32[:, :, 1:].reshape(N, MAX_ATOMS * nj)
    colst = par32[:, :, 0]
    lane_t = (jnp.arange(MAX_ATOMS * nj, dtype=jnp.int32) // nj).reshape(1, -1)
    co_flat = calculation_orders.astype(jnp.int32).T.reshape(-1)

    a = _tc_atom_proj(atom_features, w0a_t, b0)
    ag3 = _sc_gather(a, co_flat, 120).reshape(MAX_ATOMS, N, F)
    rflat = _tc_source_rounds(par_flat, colst, lane_t)
    hidx = rflat.reshape(N, MAX_ATOMS, nj).transpose(1, 0, 2).reshape(
        MAX_ATOMS, N * nj)

    hist = jnp.zeros((SLOTS, N, F), jnp.float32)
    hist = _tc_round0(hist, ag3, w1_t, b1)
    for t in range(1, MAX_ATOMS):
        g = _sc_gather(hist.reshape(SLOTS * N, F), hidx[t], 120)
        hist = _tc_round_mlp(hist, g.reshape(N, nj * F), ag3, t,
                             w0g_t, w1_t, b1)

    mem_col = membership.astype(jnp.int32).reshape(N, 1)
    soft, logits = _tc_head(hist, mem_col, gw0_t, gb0, gw1_t, gb1,
                            dw_t, db, pswap)
    shape3 = (BATCH, N_TASKS, 2)
    return (soft.reshape(shape3), logits.reshape(shape3))


# speedup vs baseline: 1.0411x; 1.0411x over previous
import functools

import jax
import jax.numpy as jnp
from jax import lax
from jax.experimental import pallas as pl
from jax.experimental.pallas import tpu as pltpu
from jax.experimental.pallas import tpu_sc as plsc

N_TASKS = 12
MAX_ATOMS = 30
N_ATOM_FEAT = 75
NGF = 30
N_OUT = 30
BATCH = 128
N = MAX_ATOMS * BATCH
F = 32
SLOTS = MAX_ATOMS + 1


def _sc_gather(table, idx, chunk):
    info = plsc.get_sparse_core_info()
    nw = info.num_cores * info.num_subcores
    b = idx.shape[0]
    bpw = b // nw
    nch = bpw // chunk
    assert bpw % chunk == 0 and chunk % 8 == 0 and chunk <= 128
    mesh = plsc.VectorSubcoreMesh(core_axis_name="c", subcore_axis_name="s")

    @functools.partial(
        pl.kernel,
        mesh=mesh,
        out_type=jax.ShapeDtypeStruct((b, F), jnp.float32),
        scratch_types=[
            pltpu.VMEM((bpw,), jnp.int32),
            pltpu.VMEM((bpw, F), jnp.float32),
            pltpu.SemaphoreType.DMA,
        ],
        compiler_params=pltpu.CompilerParams(use_tc_tiling_on_sc=False),
    )
    def k(table_hbm, idx_hbm, out_hbm, idx_v, rows_v, sem):
        wid = lax.axis_index("s") * info.num_cores + lax.axis_index("c")
        base = pl.multiple_of(wid * bpw, 8)
        pltpu.sync_copy(idx_hbm.at[pl.ds(base, bpw)], idx_v)

        def body(c, carry):
            off = pl.multiple_of(c * chunk, 8)
            pltpu.async_copy(
                table_hbm.at[idx_v.at[pl.ds(off, chunk)]],
                rows_v.at[pl.ds(off, chunk)],
                sem,
            )
            return carry

        lax.fori_loop(0, nch, body, 0)
        pltpu.make_async_copy(table_hbm.at[pl.ds(0, bpw)], rows_v, sem).wait()
        pltpu.sync_copy(rows_v, out_hbm.at[pl.ds(base, bpw)])

    return k(table, idx)


def _tc_source_rounds(par_flat, colst, lane_t):
    blk = 480
    grid = N // blk
    jt = par_flat.shape[1]

    def body(p_ref, c_ref, lt_ref, o_ref):
        i = pl.program_id(0)
        acc = jnp.full((blk, jt), MAX_ATOMS, jnp.int32)
        p = p_ref[...]
        lt = lt_ref[...]
        for tp in range(MAX_ATOMS):
            m = (p == c_ref[:, tp:tp + 1]) & (lt > tp)
            acc = jnp.where(m, tp, acc)
        rows = lax.broadcasted_iota(jnp.int32, (blk, jt), 0) + i * blk
        o_ref[...] = rows * SLOTS + acc

    return pl.pallas_call(
        body,
        grid=(grid,),
        in_specs=[
            pl.BlockSpec((blk, jt), lambda i: (i, 0)),
            pl.BlockSpec((blk, MAX_ATOMS), lambda i: (i, 0)),
            pl.BlockSpec((1, jt), lambda i: (0, 0)),
        ],
        out_specs=pl.BlockSpec((blk, jt), lambda i: (i, 0)),
        out_shape=jax.ShapeDtypeStruct((N, jt), jnp.int32),
    )(par_flat, colst, lane_t)


def _tc_round_major(rflat):
    nj = MAX_ATOMS - 1

    blk = 480
    grid = N // blk

    def body(x_ref, o_ref):
        x = x_ref[...]
        for t in range(MAX_ATOMS):
            o_ref[t] = x[:, t * nj:(t + 1) * nj]

    return pl.pallas_call(
        body,
        grid=(grid,),
        in_specs=[pl.BlockSpec((blk, MAX_ATOMS * nj), lambda i: (i, 0))],
        out_specs=pl.BlockSpec((MAX_ATOMS, blk, nj), lambda i: (0, i, 0)),
        out_shape=jax.ShapeDtypeStruct((MAX_ATOMS, N, nj), jnp.int32),
    )(rflat)

def _tc_atom_proj(x, w_t, b):
    def body(x_ref, w_ref, b_ref, o_ref):
        o_ref[...] = (
            jnp.dot(x_ref[...], w_ref[...], preferred_element_type=jnp.float32)
            + b_ref[...]
        )

    return pl.pallas_call(
        body,
        out_shape=jax.ShapeDtypeStruct((x.shape[0], F), jnp.float32),
    )(x, w_t, b)


def _tc_round_mlp(hist, gflat, ag3, t, w0g_t, w1_t, b1):
    blk = 480
    grid = N // blk

    def body(g_ref, a_ref, w0_ref, w1_ref, b1_ref, h_ref, o_ref):
        del h_ref
        h = jnp.dot(g_ref[...], w0_ref[...], preferred_element_type=jnp.float32)
        h = jnp.maximum(h + a_ref[0], 0.0)
        o = jnp.dot(h, w1_ref[...], preferred_element_type=jnp.float32)
        o_ref[:, 0, 0, :] = jnp.maximum(o + b1_ref[...], 0.0)

    return pl.pallas_call(
        body,
        grid=(grid,),
        in_specs=[
            pl.BlockSpec((blk, gflat.shape[1]), lambda i: (i, 0)),
            pl.BlockSpec((1, blk, F), lambda i, t=t: (t, i, 0)),
            pl.BlockSpec(w0g_t.shape, lambda i: (0, 0)),
            pl.BlockSpec(w1_t.shape, lambda i: (0, 0)),
            pl.BlockSpec(b1.shape, lambda i: (0, 0)),
            pl.BlockSpec((blk, 1, 1, F), lambda i, t=t: (i, t, 0, 0)),
        ],
        out_specs=pl.BlockSpec((blk, 1, 1, F), lambda i, t=t: (i, t, 0, 0)),
        out_shape=jax.ShapeDtypeStruct((N, SLOTS, 1, F), jnp.float32),
        input_output_aliases={5: 0},
    )(gflat, ag3, w0g_t, w1_t, b1, hist)


def _tc_round0(hist, ag3, w1_t, b1):
    blk = 480
    grid = N // blk

    def body(a_ref, w1_ref, b1_ref, h_ref, o_ref):
        del h_ref
        h = jnp.maximum(a_ref[0], 0.0)
        o = jnp.dot(h, w1_ref[...], preferred_element_type=jnp.float32)
        o_ref[:, 0, 0, :] = jnp.maximum(o + b1_ref[...], 0.0)

    return pl.pallas_call(
        body,
        grid=(grid,),
        in_specs=[
            pl.BlockSpec((1, blk, F), lambda i: (0, i, 0)),
            pl.BlockSpec(w1_t.shape, lambda i: (0, 0)),
            pl.BlockSpec(b1.shape, lambda i: (0, 0)),
            pl.BlockSpec((blk, 1, 1, F), lambda i: (i, 0, 0, 0)),
        ],
        out_specs=pl.BlockSpec((blk, 1, 1, F), lambda i: (i, 0, 0, 0)),
        out_shape=jax.ShapeDtypeStruct((N, SLOTS, 1, F), jnp.float32),
        input_output_aliases={3: 0},
    )(ag3, w1_t, b1, hist)


def _tc_head(hist, mem_col, gw0_t, gb0, gw1_t, gb1, dw_t, db, pswap):
    def body(x_ref, m_ref, w0_ref, b0_ref, w1_ref, b1_ref, wd_ref, bd_ref,
             p_ref, soft_ref, logit_ref):
        seg = lax.broadcasted_iota(jnp.int32, (N, BATCH), 1)
        oh = (m_ref[...] == seg).astype(jnp.float32)
        g = lax.dot_general(
            oh, x_ref[:, 0, 0, :], (((0,), (0,)), ((), ())),
            preferred_element_type=jnp.float32,
        )
        h = jnp.maximum(
            jnp.dot(g, w0_ref[...], preferred_element_type=jnp.float32)
            + b0_ref[...], 0.0)
        h = jnp.maximum(
            jnp.dot(h, w1_ref[...], preferred_element_type=jnp.float32)
            + b1_ref[...], 0.0)
        x = (jnp.dot(h, wd_ref[...], preferred_element_type=jnp.float32)
             + bd_ref[...])
        partner = jnp.dot(x, p_ref[...], preferred_element_type=jnp.float32)
        m = jnp.maximum(x, partner)
        e = jnp.exp(x - m)
        s = e + jnp.exp(partner - m)
        soft_ref[...] = e / s
        logit_ref[...] = x

    def _full(x):
        zero = tuple(0 for _ in x.shape)
        return pl.BlockSpec(x.shape, lambda i, _z=zero: _z)

    specs = [pl.BlockSpec((N, 1, 1, F), lambda i: (0, MAX_ATOMS - 1, 0, 0))]
    specs += [_full(x)
              for x in (mem_col, gw0_t, gb0, gw1_t, gb1, dw_t, db, pswap)]
    oshape = (BATCH, 2 * N_TASKS)
    return pl.pallas_call(
        body,
        grid=(1,),
        in_specs=specs,
        out_specs=(pl.BlockSpec(oshape, lambda i: (0, 0)),
                   pl.BlockSpec(oshape, lambda i: (0, 0))),
        out_shape=(
            jax.ShapeDtypeStruct((BATCH, 2 * N_TASKS), jnp.float32),
            jax.ShapeDtypeStruct((BATCH, 2 * N_TASKS), jnp.float32),
        ),
    )(hist, mem_col, gw0_t, gb0, gw1_t, gb1, dw_t, db, pswap)


def kernel(atom_features, parents, calculation_orders, calculation_masks,
           membership, n_atoms, dag_W0, dag_b0, dag_W1, dag_b1,
           gat_W0, gat_b0, gat_W1, gat_b1, dense_W, dense_b):
    del calculation_masks, n_atoms

    w0a_t = dag_W0[:, :N_ATOM_FEAT].T
    b0 = dag_b0.reshape(1, F)
    w0g = dag_W0[:, N_ATOM_FEAT:].reshape(F, MAX_ATOMS - 1, NGF)
    w0g = jnp.pad(w0g, ((0, 0), (0, 0), (0, F - NGF)))
    w0g_t = w0g.reshape(F, (MAX_ATOMS - 1) * F).T
    w1_t = jnp.pad(dag_W1.T, ((0, 0), (0, F - N_OUT)))
    b1 = jnp.pad(dag_b1, (0, F - N_OUT)).reshape(1, F)
    gw0_t = jnp.pad(gat_W0.T, ((0, F - NGF), (0, 0)))
    gb0 = gat_b0.reshape(1, -1)
    gw1_t = jnp.pad(gat_W1.T, ((0, 0), (0, F - N_OUT)))
    gb1 = jnp.pad(gat_b1, (0, F - N_OUT)).reshape(1, F)
    dw_t = jnp.pad(dense_W.T, ((0, F - N_OUT), (0, 0)))
    db = dense_b.reshape(1, -1)
    ncls = 2 * N_TASKS
    pair = jnp.arange(ncls)
    pswap = (pair[:, None] == (pair ^ 1)[None, :]).astype(jnp.float32)

    par32 = parents.astype(jnp.int32)
    nj = MAX_ATOMS - 1
    par_flat = par32[:, :, 1:].reshape(N, MAX_ATOMS * nj)
    colst = par32[:, :, 0]
    lane_t = (jnp.arange(MAX_ATOMS * nj, dtype=jnp.int32) // nj).reshape(1, -1)
    co_flat = calculation_orders.astype(jnp.int32).T.reshape(-1)

    a = _tc_atom_proj(atom_features, w0a_t, b0)
    ag3 = _sc_gather(a, co_flat, 120).reshape(MAX_ATOMS, N, F)
    rflat = _tc_source_rounds(par_flat, colst, lane_t)
    hidx = _tc_round_major(rflat).reshape(MAX_ATOMS, N * nj)

    hist = jnp.zeros((N, SLOTS, 1, F), jnp.float32)
    hist = _tc_round0(hist, ag3, w1_t, b1)
    for t in range(1, MAX_ATOMS):
        g = _sc_gather(hist.reshape(SLOTS * N, F), hidx[t], 120)
        hist = _tc_round_mlp(hist, g.reshape(N, nj * F), ag3, t,
                             w0g_t, w1_t, b1)

    mem_col = membership.astype(jnp.int32).reshape(N, 1)
    soft, logits = _tc_head(hist, mem_col, gw0_t, gb0, gw1_t, gb1,
                            dw_t, db, pswap)
    shape3 = (BATCH, N_TASKS, 2)
    return (soft.reshape(shape3), logits.reshape(shape3))


# speedup vs baseline: 1.7097x; 1.6422x over previous
import functools

import jax
import jax.numpy as jnp
from jax import lax
from jax.experimental import pallas as pl
from jax.experimental.pallas import tpu as pltpu
from jax.experimental.pallas import tpu_sc as plsc
from jax._src.pallas import mpmd as _mpmd

N_TASKS = 12
MAX_ATOMS = 30
N_ATOM_FEAT = 75
NGF = 30
N_OUT = 30
BATCH = 128
N = MAX_ATOMS * BATCH
F = 32
SLOTS = MAX_ATOMS + 1


def _sc_gather(table, idx, chunk):
    info = plsc.get_sparse_core_info()
    nw = info.num_cores * info.num_subcores
    b = idx.shape[0]
    bpw = b // nw
    nch = bpw // chunk
    assert bpw % chunk == 0 and chunk % 8 == 0 and chunk <= 128
    mesh = plsc.VectorSubcoreMesh(core_axis_name="c", subcore_axis_name="s")

    @functools.partial(
        pl.kernel,
        mesh=mesh,
        out_type=jax.ShapeDtypeStruct((b, F), jnp.float32),
        scratch_types=[
            pltpu.VMEM((bpw,), jnp.int32),
            pltpu.VMEM((bpw, F), jnp.float32),
            pltpu.SemaphoreType.DMA,
        ],
        compiler_params=pltpu.CompilerParams(use_tc_tiling_on_sc=False),
    )
    def k(table_hbm, idx_hbm, out_hbm, idx_v, rows_v, sem):
        wid = lax.axis_index("s") * info.num_cores + lax.axis_index("c")
        base = pl.multiple_of(wid * bpw, 8)
        pltpu.sync_copy(idx_hbm.at[pl.ds(base, bpw)], idx_v)

        def body(c, carry):
            off = pl.multiple_of(c * chunk, 8)
            pltpu.async_copy(
                table_hbm.at[idx_v.at[pl.ds(off, chunk)]],
                rows_v.at[pl.ds(off, chunk)],
                sem,
            )
            return carry

        lax.fori_loop(0, nch, body, 0)
        pltpu.make_async_copy(table_hbm.at[pl.ds(0, bpw)], rows_v, sem).wait()
        pltpu.sync_copy(rows_v, out_hbm.at[pl.ds(base, bpw)])

    return k(table, idx)


def _sc_scatter(state, vals, idx):
    info = plsc.get_sparse_core_info()
    nw = info.num_cores * info.num_subcores
    n = vals.shape[0]
    rpw = n // nw
    assert n % nw == 0 and rpw <= 128 and rpw % 8 == 0
    mesh = plsc.VectorSubcoreMesh(core_axis_name="c", subcore_axis_name="s")

    def body(state_hbm, vals_hbm, idx_hbm, out_hbm, idx_v, rows_v, sem):
        del state_hbm
        wid = lax.axis_index("s") * info.num_cores + lax.axis_index("c")
        base = pl.multiple_of(wid * rpw, 8)
        h1 = pltpu.async_copy(idx_hbm.at[pl.ds(base, rpw)], idx_v, sem)
        h2 = pltpu.async_copy(vals_hbm.at[pl.ds(base, rpw)], rows_v, sem)
        h1.wait()
        h2.wait()
        pltpu.async_copy(rows_v, out_hbm.at[idx_v], sem).wait()

    k = _mpmd._mpmd_map(
        [(mesh, body)],
        jax.ShapeDtypeStruct(state.shape, jnp.float32),
        input_output_aliases={0: 0},
        scratch_types=[
            pltpu.VMEM((rpw,), jnp.int32),
            pltpu.VMEM((rpw, F), jnp.float32),
            pltpu.SemaphoreType.DMA,
        ],
        compiler_params=pltpu.CompilerParams(use_tc_tiling_on_sc=False),
    )
    return k(state, vals, idx)


def _tc_atom_proj(x, w_t, b):
    def body(x_ref, w_ref, b_ref, o_ref):
        o_ref[...] = (
            jnp.dot(x_ref[...], w_ref[...], preferred_element_type=jnp.float32)
            + b_ref[...]
        )

    return pl.pallas_call(
        body,
        out_shape=jax.ShapeDtypeStruct((x.shape[0], F), jnp.float32),
    )(x, w_t, b)


def _tc_round_mlp(gflat, ag, w0g_t, w1_t, b1):
    blk = 480
    grid = N // blk

    def body(g_ref, a_ref, w0_ref, w1_ref, b1_ref, o_ref):
        h = jnp.dot(g_ref[...], w0_ref[...], preferred_element_type=jnp.float32)
        h = jnp.maximum(h + a_ref[...], 0.0)
        o = jnp.dot(h, w1_ref[...], preferred_element_type=jnp.float32)
        o_ref[...] = jnp.maximum(o + b1_ref[...], 0.0)

    return pl.pallas_call(
        body,
        grid=(grid,),
        in_specs=[
            pl.BlockSpec((blk, gflat.shape[1]), lambda i: (i, 0)),
            pl.BlockSpec((blk, F), lambda i: (i, 0)),
            pl.BlockSpec(w0g_t.shape, lambda i: (0, 0)),
            pl.BlockSpec(w1_t.shape, lambda i: (0, 0)),
            pl.BlockSpec(b1.shape, lambda i: (0, 0)),
        ],
        out_specs=pl.BlockSpec((blk, F), lambda i: (i, 0)),
        out_shape=jax.ShapeDtypeStruct((N, F), jnp.float32),
    )(gflat, ag, w0g_t, w1_t, b1)


def _tc_head(last_out, mem_col, gw0_t, gb0, gw1_t, gb1, dw_t, db, pswap):
    def body(x_ref, m_ref, w0_ref, b0_ref, w1_ref, b1_ref, wd_ref, bd_ref,
             p_ref, soft_ref, logit_ref):
        seg = lax.broadcasted_iota(jnp.int32, (N, BATCH), 1)
        oh = (m_ref[...] == seg).astype(jnp.float32)
        g = lax.dot_general(
            oh, x_ref[...], (((0,), (0,)), ((), ())),
            preferred_element_type=jnp.float32,
        )
        h = jnp.maximum(
            jnp.dot(g, w0_ref[...], preferred_element_type=jnp.float32)
            + b0_ref[...], 0.0)
        h = jnp.maximum(
            jnp.dot(h, w1_ref[...], preferred_element_type=jnp.float32)
            + b1_ref[...], 0.0)
        x = (jnp.dot(h, wd_ref[...], preferred_element_type=jnp.float32)
             + bd_ref[...])
        partner = jnp.dot(x, p_ref[...], preferred_element_type=jnp.float32)
        m = jnp.maximum(x, partner)
        e = jnp.exp(x - m)
        s = e + jnp.exp(partner - m)
        soft_ref[...] = e / s
        logit_ref[...] = x

    return pl.pallas_call(
        body,
        out_shape=(
            jax.ShapeDtypeStruct((BATCH, 2 * N_TASKS), jnp.float32),
            jax.ShapeDtypeStruct((BATCH, 2 * N_TASKS), jnp.float32),
        ),
    )(last_out, mem_col, gw0_t, gb0, gw1_t, gb1, dw_t, db, pswap)


def kernel(atom_features, parents, calculation_orders, calculation_masks,
           membership, n_atoms, dag_W0, dag_b0, dag_W1, dag_b1,
           gat_W0, gat_b0, gat_W1, gat_b1, dense_W, dense_b):
    del calculation_masks, n_atoms

    w0a_t = dag_W0[:, :N_ATOM_FEAT].T
    b0 = dag_b0.reshape(1, F)
    w0g = dag_W0[:, N_ATOM_FEAT:].reshape(F, MAX_ATOMS - 1, NGF)
    w0g = jnp.pad(w0g, ((0, 0), (0, 0), (0, F - NGF)))
    w0g_t = w0g.reshape(F, (MAX_ATOMS - 1) * F).T
    w1_t = jnp.pad(dag_W1.T, ((0, 0), (0, F - N_OUT)))
    b1 = jnp.pad(dag_b1, (0, F - N_OUT)).reshape(1, F)
    gw0_t = jnp.pad(gat_W0.T, ((0, F - NGF), (0, 0)))
    gb0 = gat_b0.reshape(1, -1)
    gw1_t = jnp.pad(gat_W1.T, ((0, 0), (0, F - N_OUT)))
    gb1 = jnp.pad(gat_b1, (0, F - N_OUT)).reshape(1, F)
    dw_t = jnp.pad(dense_W.T, ((0, F - N_OUT), (0, 0)))
    db = dense_b.reshape(1, -1)
    ncls = 2 * N_TASKS
    pair = jnp.arange(ncls)
    pswap = (pair[:, None] == (pair ^ 1)[None, :]).astype(jnp.float32)

    rows31 = (jnp.arange(N, dtype=jnp.int32) * SLOTS)[None, :, None]
    par_t = parents.astype(jnp.int32).transpose(1, 0, 2)
    gidx = (par_t[:, :, 1:] + rows31).reshape(MAX_ATOMS, N * (MAX_ATOMS - 1))
    sidx = par_t[:, :, 0] + rows31[:, :, 0]
    co_flat = calculation_orders.astype(jnp.int32).T.reshape(-1)

    a = _tc_atom_proj(atom_features, w0a_t, b0)
    ag = _sc_gather(a, co_flat, 120).reshape(MAX_ATOMS, N, F)

    state = jnp.zeros((N * SLOTS, F), jnp.float32)
    out_t = None
    for t in range(MAX_ATOMS):
        g = _sc_gather(state, gidx[t], 120)
        out_t = _tc_round_mlp(g.reshape(N, (MAX_ATOMS - 1) * F), ag[t],
                              w0g_t, w1_t, b1)
        state = _sc_scatter(state, out_t, sidx[t])

    mem_col = membership.astype(jnp.int32).reshape(N, 1)
    soft, logits = _tc_head(out_t, mem_col, gw0_t, gb0, gw1_t, gb1,
                            dw_t, db, pswap)
    shape3 = (BATCH, N_TASKS, 2)
    return (soft.reshape(shape3), logits.reshape(shape3))
